# baseline (speedup 1.0000x reference)
import jax
import jax.numpy as jnp
from jax.experimental import pallas as pl
from jax.experimental.pallas import tpu as pltpu

EPS = 1e-5
_B = 8192

_DN = (((1,), (0,)), ((), ()))


def _stats_kernel(n_ref, xyzT_ref, featT_ref, w3_ref, wf_ref, g_ref, b_ref,
                  ws3_ref, wsf_ref, sh_ref, s_ref, q_ref):
    i = pl.program_id(0)
    nb = pl.num_programs(0)
    n = n_ref[0]
    a = xyzT_ref[...]
    f = featT_ref[...]
    p = jax.lax.dot_general(w3_ref[...], a, _DN,
                            preferred_element_type=jnp.float32)
    p = p + jax.lax.dot_general(wf_ref[...], f, _DN,
                                preferred_element_type=jnp.float32)

    def _acc(ps, pq):
        @pl.when(i == 0)
        def _():
            s_ref[...] = ps
            q_ref[...] = pq

        @pl.when(i != 0)
        def _():
            s_ref[...] += ps
            q_ref[...] += pq

    @pl.when(i != nb - 1)
    def _full():
        _acc(jnp.sum(p, axis=1, keepdims=True),
             jnp.sum(p * p, axis=1, keepdims=True))

    @pl.when(i == nb - 1)
    def _last():
        lane = jax.lax.broadcasted_iota(jnp.int32, (16, _B), 1)
        pm = jnp.where(lane < n - i * _B, p, 0.0)
        _acc(jnp.sum(pm, axis=1, keepdims=True),
             jnp.sum(pm * pm, axis=1, keepdims=True))
        nf = n.astype(jnp.float32)
        mean = s_ref[...] / nf
        var = q_ref[...] / nf - mean * mean
        scale = g_ref[...] * jax.lax.rsqrt(var + EPS)
        ws3_ref[...] = w3_ref[...] * scale
        wsf_ref[...] = wf_ref[...] * scale
        sh_ref[...] = b_ref[...] - mean * scale


def _apply_kernel(xyzT_ref, featT_ref, ws3_ref, wsf_ref, sh_ref,
                  outT_ref, velT_ref):
    a = xyzT_ref[...]
    f = featT_ref[...]
    velT_ref[...] = f
    p = jax.lax.dot_general(ws3_ref[...], a, _DN,
                            preferred_element_type=jnp.float32)
    p = p + jax.lax.dot_general(wsf_ref[...], f, _DN,
                                preferred_element_type=jnp.float32)
    outT_ref[...] = jnp.maximum(p + sh_ref[...], 0.0)


def kernel(xyz, feat, offset, velocities, W, gamma, beta):
    n = xyz.shape[0]
    nb = pl.cdiv(n, _B)
    xyzT = xyz.T
    featT = feat.T
    w3 = W[:, :3]
    wf = W[:, 3:]
    g = gamma.reshape(16, 1)
    b = beta.reshape(16, 1)
    n_arr = jnp.full((1,), n, dtype=jnp.int32)

    const = lambda i: (0, 0)
    row = lambda i: (0, i)

    ws3, wsf, sh = pl.pallas_call(
        _stats_kernel,
        grid=(nb,),
        in_specs=[
            pl.BlockSpec(memory_space=pltpu.SMEM),
            pl.BlockSpec((3, _B), row),
            pl.BlockSpec((13, _B), row),
            pl.BlockSpec((16, 3), const),
            pl.BlockSpec((16, 13), const),
            pl.BlockSpec((16, 1), const),
            pl.BlockSpec((16, 1), const),
        ],
        out_specs=[
            pl.BlockSpec((16, 3), const),
            pl.BlockSpec((16, 13), const),
            pl.BlockSpec((16, 1), const),
        ],
        out_shape=[
            jax.ShapeDtypeStruct((16, 3), jnp.float32),
            jax.ShapeDtypeStruct((16, 13), jnp.float32),
            jax.ShapeDtypeStruct((16, 1), jnp.float32),
        ],
        scratch_shapes=[
            pltpu.VMEM((16, 1), jnp.float32),
            pltpu.VMEM((16, 1), jnp.float32),
        ],
    )(n_arr, xyzT, featT, w3, wf, g, b)

    outT, velT = pl.pallas_call(
        _apply_kernel,
        grid=(nb,),
        in_specs=[
            pl.BlockSpec((3, _B), row),
            pl.BlockSpec((13, _B), row),
            pl.BlockSpec((16, 3), const),
            pl.BlockSpec((16, 13), const),
            pl.BlockSpec((16, 1), const),
        ],
        out_specs=[
            pl.BlockSpec((16, _B), row),
            pl.BlockSpec((13, _B), row),
        ],
        out_shape=[
            jax.ShapeDtypeStruct((16, n), jnp.float32),
            jax.ShapeDtypeStruct((13, n), jnp.float32),
        ],
    )(xyzT, featT, ws3, wsf, sh)

    return (xyz, outT.T, offset, velT.T)
